# baseline (speedup 1.0000x reference)
import functools

import jax
import jax.numpy as jnp
from jax import lax
from jax.experimental import pallas as pl
from jax.experimental.pallas import tpu as pltpu
from jax.experimental.pallas import tpu_sc as plsc

LANES = 16


def _make_kernel(batch, seq, dim):
    info = plsc.get_sparse_core_info()
    nc, ns = info.num_cores, info.num_subcores
    nw = nc * ns
    seq_per_w = seq // nw
    cs = 16
    nchunk = seq_per_w // cs
    vecs_per_row = dim // LANES

    mesh = plsc.VectorSubcoreMesh(core_axis_name="c", subcore_axis_name="s")

    @functools.partial(
        pl.kernel,
        mesh=mesh,
        out_type=jax.ShapeDtypeStruct((batch, seq, dim), jnp.float32),
        scratch_types=[
            pltpu.VMEM((cs, dim), jnp.float32),
            pltpu.VMEM((cs, dim), jnp.float32),
        ],
    )
    def k(in_hbm, pos_hbm, out_hbm, pos_v, buf_v):
        wid = lax.axis_index("s") * nc + lax.axis_index("c")
        seq0 = wid * seq_per_w

        def chunk_body(ci, _):
            s0 = seq0 + ci * cs
            pltpu.sync_copy(pos_hbm.at[pl.ds(s0, cs)], pos_v)

            def batch_body(b, __):
                pltpu.sync_copy(in_hbm.at[b, pl.ds(s0, cs)], buf_v)

                def row_body(r, ___):
                    def col_body(cc, ____):
                        sl = pl.ds(cc * LANES, LANES)
                        buf_v[r, sl] = buf_v[r, sl] + pos_v[r, sl]
                        return ____

                    return lax.fori_loop(0, vecs_per_row, col_body, ___)

                lax.fori_loop(0, cs, row_body, 0)
                pltpu.sync_copy(buf_v, out_hbm.at[b, pl.ds(s0, cs)])
                return __

            lax.fori_loop(0, batch, batch_body, 0)
            return _

        lax.fori_loop(0, nchunk, chunk_body, 0)

    return k


def kernel(inputs, position_embeddings):
    batch, seq, dim = inputs.shape
    k = _make_kernel(batch, seq, dim)
    return k(inputs, position_embeddings)


# speedup vs baseline: 2.4674x; 2.4674x over previous
import functools

import jax
import jax.numpy as jnp
from jax import lax
from jax.experimental import pallas as pl
from jax.experimental.pallas import tpu as pltpu
from jax.experimental.pallas import tpu_sc as plsc

LANES = 16
CS = 8


def _make_kernel(batch, seq, dim):
    info = plsc.get_sparse_core_info()
    nc, ns = info.num_cores, info.num_subcores
    nw = nc * ns
    seq_per_w = seq // nw
    nchunk = seq_per_w // CS
    npair = nchunk // 2
    vecs_per_row = dim // LANES

    mesh = plsc.VectorSubcoreMesh(core_axis_name="c", subcore_axis_name="s")

    buf_t = pltpu.VMEM((CS, dim), jnp.float32)

    @functools.partial(
        pl.kernel,
        mesh=mesh,
        out_type=jax.ShapeDtypeStruct((batch, seq, dim), jnp.float32),
        scratch_types=(
            [buf_t] * 4
            + [buf_t] * 4
            + [buf_t, buf_t]
            + [pltpu.SemaphoreType.DMA] * 4
        ),
    )
    def k(in_hbm, pos_hbm, out_hbm, *scratch):
        bufs_a = scratch[0:4]
        bufs_b = scratch[4:8]
        pos_a, pos_b = scratch[8], scratch[9]
        sem_in_a, sem_in_b, sem_out_a, sem_out_b = scratch[10:14]

        wid = lax.axis_index("s") * nc + lax.axis_index("c")
        seq0 = wid * seq_per_w

        def issue_loads(s0, bufs, pos_v, sem):
            pltpu.async_copy(pos_hbm.at[pl.ds(s0, CS)], pos_v, sem)
            for b in range(batch):
                pltpu.async_copy(in_hbm.at[b, pl.ds(s0, CS)], bufs[b], sem)

        def wait_loads(s0, bufs, pos_v, sem):
            pltpu.make_async_copy(pos_hbm.at[pl.ds(s0, CS)], pos_v, sem).wait()
            for b in range(batch):
                pltpu.make_async_copy(
                    in_hbm.at[b, pl.ds(s0, CS)], bufs[b], sem).wait()

        def add_and_store(s0, bufs, pos_v, sem_out):
            for b in range(batch):
                buf = bufs[b]

                def row_body(r, _):
                    for c in range(vecs_per_row):
                        sl = pl.ds(c * LANES, LANES)
                        plsc.addupdate(buf.at[r, sl], pos_v[r, sl])
                    return _

                lax.fori_loop(0, CS, row_body, 0)
                pltpu.async_copy(buf, out_hbm.at[b, pl.ds(s0, CS)], sem_out)

        def wait_stores(s0, bufs, sem_out):
            for b in range(batch):
                pltpu.make_async_copy(
                    bufs[b], out_hbm.at[b, pl.ds(s0, CS)], sem_out).wait()

        issue_loads(seq0, bufs_a, pos_a, sem_in_a)

        def pair_body(ci, _):
            s0 = seq0 + (2 * ci) * CS
            s1 = s0 + CS

            @pl.when(ci > 0)
            def _w():
                wait_stores(s1, bufs_b, sem_out_b)

            issue_loads(s1, bufs_b, pos_b, sem_in_b)

            wait_loads(s0, bufs_a, pos_a, sem_in_a)
            add_and_store(s0, bufs_a, pos_a, sem_out_a)

            wait_loads(s1, bufs_b, pos_b, sem_in_b)
            add_and_store(s1, bufs_b, pos_b, sem_out_b)

            wait_stores(s0, bufs_a, sem_out_a)

            @pl.when(ci < npair - 1)
            def _r():
                issue_loads(s0 + 2 * CS, bufs_a, pos_a, sem_in_a)

            return _

        lax.fori_loop(0, npair, pair_body, 0)

        wait_stores(seq0 + (nchunk - 1) * CS, bufs_b, sem_out_b)

    return k


def kernel(inputs, position_embeddings):
    batch, seq, dim = inputs.shape
    k = _make_kernel(batch, seq, dim)
    return k(inputs, position_embeddings)


# speedup vs baseline: 2.7050x; 1.0963x over previous
import functools

import jax
import jax.numpy as jnp
from jax import lax
from jax.experimental import pallas as pl
from jax.experimental.pallas import tpu as pltpu
from jax.experimental.pallas import tpu_sc as plsc

LANES = 16
CS = 8


def _make_kernel(batch, seq, dim):
    info = plsc.get_sparse_core_info()
    nc, ns = info.num_cores, info.num_subcores
    nw = nc * ns
    seq_per_w = seq // nw
    nchunk = seq_per_w // CS
    npair = nchunk // 2
    vecs_per_row = dim // LANES

    mesh = plsc.VectorSubcoreMesh(core_axis_name="c", subcore_axis_name="s")

    buf_t = pltpu.VMEM((CS, dim), jnp.float32)

    @functools.partial(
        pl.kernel,
        mesh=mesh,
        out_type=jax.ShapeDtypeStruct((batch, seq, dim), jnp.float32),
        scratch_types=(
            [buf_t] * 4
            + [buf_t] * 4
            + [buf_t, buf_t]
            + [pltpu.SemaphoreType.DMA] * 2
            + [pltpu.SemaphoreType.DMA] * 8
        ),
    )
    def k(in_hbm, pos_hbm, out_hbm, *scratch):
        bufs = (scratch[0:4], scratch[4:8])
        pos_v = (scratch[8], scratch[9])
        sem_in = (scratch[10], scratch[11])
        sem_out = (scratch[12:16], scratch[16:20])

        wid = lax.axis_index("s") * nc + lax.axis_index("c")
        seq0 = wid * seq_per_w

        def issue_loads(s0, st):
            pltpu.async_copy(pos_hbm.at[pl.ds(s0, CS)], pos_v[st], sem_in[st])
            for b in range(batch):
                pltpu.async_copy(
                    in_hbm.at[b, pl.ds(s0, CS)], bufs[st][b], sem_in[st])

        def wait_loads(s0, st):
            pltpu.make_async_copy(
                pos_hbm.at[pl.ds(s0, CS)], pos_v[st], sem_in[st]).wait()
            for b in range(batch):
                pltpu.make_async_copy(
                    in_hbm.at[b, pl.ds(s0, CS)], bufs[st][b], sem_in[st]).wait()

        def add_rows(st, b):
            buf = bufs[st][b]
            pv = pos_v[st]

            def row_body(r, _):
                for c in range(vecs_per_row):
                    sl = pl.ds(c * LANES, LANES)
                    plsc.addupdate(buf.at[r, sl], pv[r, sl])
                return _

            lax.fori_loop(0, CS, row_body, 0)

        def phase(s0, st, reload_pred, first):
            so = st ^ 1
            wait_loads(s0, st)
            for b in range(batch):
                add_rows(st, b)
                pltpu.async_copy(
                    bufs[st][b], out_hbm.at[b, pl.ds(s0, CS)], sem_out[st][b])

                @pl.when(reload_pred)
                def _r():
                    if not first:
                        pltpu.make_async_copy(
                            bufs[so][b], out_hbm.at[b, pl.ds(s0, CS)],
                            sem_out[so][b]).wait()
                    pltpu.async_copy(
                        in_hbm.at[b, pl.ds(s0 + CS, CS)], bufs[so][b],
                        sem_in[so])

        issue_loads(seq0, 0)
        issue_loads(seq0 + CS, 1)

        def pair_body(ci, _):
            s0 = seq0 + (2 * ci) * CS
            s1 = s0 + CS

            @pl.when(ci > 0)
            def _pb():
                pltpu.async_copy(
                    pos_hbm.at[pl.ds(s1, CS)], pos_v[1], sem_in[1])

            phase(s0, 0, ci > 0, first=False)

            @pl.when(ci < npair - 1)
            def _pa():
                pltpu.async_copy(
                    pos_hbm.at[pl.ds(s1 + CS, CS)], pos_v[0], sem_in[0])

            phase(s1, 1, ci < npair - 1, first=False)
            return _

        lax.fori_loop(0, npair, pair_body, 0)

        tail = seq0 + (nchunk - 2) * CS
        for st in range(2):
            for b in range(batch):
                pltpu.make_async_copy(
                    bufs[st][b],
                    out_hbm.at[b, pl.ds(tail + st * CS, CS)],
                    sem_out[st][b]).wait()

    return k


def kernel(inputs, position_embeddings):
    batch, seq, dim = inputs.shape
    k = _make_kernel(batch, seq, dim)
    return k(inputs, position_embeddings)
